# baseline (speedup 1.0000x reference)
import functools

import jax
import jax.numpy as jnp
from jax import lax
from jax.experimental import pallas as pl
from jax.experimental.pallas import tpu as pltpu
from jax.experimental.pallas import tpu_sc as plsc

VOCAB = 1000000
N_PROMPT = 64
D = 64
DP = 128
REPLACING_ID = 0
LANES = 16
NC, NS = 2, 16
NW = NC * NS
CHUNK = 128
NBUF = 6


def _make_kernel(n_tok: int):
  assert n_tok % (NW * CHUNK) == 0
  per_w = n_tok // NW
  n_groups = per_w // LANES
  n_chunks = per_w // CHUNK

  mesh = plsc.VectorSubcoreMesh(core_axis_name="c", subcore_axis_name="s")

  @functools.partial(
      pl.kernel,
      out_type=jax.ShapeDtypeStruct((n_tok, D), jnp.float32),
      mesh=mesh,
      compiler_params=pltpu.CompilerParams(
          needs_layout_passes=False, use_tc_tiling_on_sc=False),
      scratch_types=[
          pltpu.VMEM((per_w,), jnp.int32),
          pltpu.VMEM((per_w,), jnp.int32),
          pltpu.VMEM((NBUF, CHUNK, DP), jnp.float32),
          pltpu.VMEM((LANES, D), jnp.float32),
          pltpu.SemaphoreType.DMA((NBUF,)),
          pltpu.SemaphoreType.DMA((NBUF,)),
          pltpu.SemaphoreType.DMA,
      ],
  )
  def k(ids_hbm, table_hbm, prompt_hbm, out_hbm, ids_v, idx_v, rows_v,
        prow_v, g_sems, o_sems, p_sem):
    wid = lax.axis_index("s") * NC + lax.axis_index("c")
    base = wid * per_w
    lane = lax.iota(jnp.int32, LANES)

    pltpu.sync_copy(ids_hbm.at[pl.ds(base, per_w)], ids_v)

    def p1(j, pacc):
      ids = ids_v[pl.ds(j * LANES, LANES)]
      is_p = ids >= VOCAB
      idx_v[pl.ds(j * LANES, LANES)] = jnp.where(is_p, REPLACING_ID, ids)
      return pacc + is_p.astype(jnp.int32)

    pacc = lax.fori_loop(0, n_groups, p1, jnp.zeros((LANES,), jnp.int32))
    n_prompt = jnp.sum(pacc)

    def _gather(c, b):
      return pltpu.make_async_copy(
          table_hbm.at[idx_v.at[pl.ds(c * CHUNK, CHUNK)]],
          rows_v.at[b], g_sems.at[b])

    def _out(c, b):
      return pltpu.make_async_copy(
          rows_v.at[b, :, pl.ds(0, D)],
          out_hbm.at[pl.ds(base + c * CHUNK, CHUNK)], o_sems.at[b])

    for c in range(n_chunks):
      b = c % NBUF
      if c >= NBUF:
        _out(c - NBUF, b).wait()
      _gather(c, b).start()
      if c >= 1:
        pb = (c - 1) % NBUF
        _gather(c - 1, pb).wait()
        _out(c - 1, pb).start()
    lb = (n_chunks - 1) % NBUF
    _gather(n_chunks - 1, lb).wait()
    _out(n_chunks - 1, lb).start()
    for c in range(n_chunks - NBUF + 1, n_chunks):
      _out(c, c % NBUF).wait()
    _out(n_chunks - NBUF, (n_chunks - NBUF) % NBUF).wait()

    @pl.when(n_prompt > 0)
    def _prompt_pass():
      @pl.loop(0, n_groups)
      def _p3(j):
        ids = ids_v[pl.ds(j * LANES, LANES)]
        is_p = ids >= VOCAB

        @pl.when(jnp.sum(is_p.astype(jnp.int32)) > 0)
        def _fixup():
          rel = jnp.minimum(ids - VOCAB, N_PROMPT - 1)
          pos = base + j * LANES + lane
          combo = jnp.where(is_p, pos * N_PROMPT + rel, -1)
          fill = jnp.max(combo)
          rel16 = jnp.where(is_p, rel, fill & (N_PROMPT - 1))
          pos16 = jnp.where(is_p, pos, fill >> 6)
          pltpu.async_copy(prompt_hbm.at[rel16], prow_v, p_sem).wait()
          pltpu.async_copy(prow_v, out_hbm.at[pos16], p_sem).wait()

  return k


def kernel(input_ids, labels, table, prompt_table):
  B, L = input_ids.shape
  n_tok = B * L
  ids_flat = input_ids.reshape(n_tok).astype(jnp.int32)
  table_p = jnp.pad(table, ((0, 0), (0, DP - D)))
  out = _make_kernel(n_tok)(ids_flat, table_p, prompt_table)
  return out.reshape(B, L, D)


# speedup vs baseline: 1.0138x; 1.0138x over previous
import functools

import jax
import jax.numpy as jnp
from jax import lax
from jax.experimental import pallas as pl
from jax.experimental.pallas import tpu as pltpu
from jax.experimental.pallas import tpu_sc as plsc

VOCAB = 1000000
N_PROMPT = 64
D = 64
DP = 128
REPLACING_ID = 0
LANES = 16
NC, NS = 2, 16
NW = NC * NS
CHUNK = 256
NBUF = 3


def _make_kernel(n_tok: int):
  assert n_tok % (NW * CHUNK) == 0
  per_w = n_tok // NW
  n_groups = per_w // LANES
  n_chunks = per_w // CHUNK

  mesh = plsc.VectorSubcoreMesh(core_axis_name="c", subcore_axis_name="s")

  @functools.partial(
      pl.kernel,
      out_type=jax.ShapeDtypeStruct((n_tok, D), jnp.float32),
      mesh=mesh,
      compiler_params=pltpu.CompilerParams(
          needs_layout_passes=False, use_tc_tiling_on_sc=False),
      scratch_types=[
          pltpu.VMEM((per_w,), jnp.int32),
          pltpu.VMEM((per_w,), jnp.int32),
          pltpu.VMEM((NBUF, CHUNK, DP), jnp.float32),
          pltpu.VMEM((LANES, D), jnp.float32),
          pltpu.SemaphoreType.DMA((NBUF,)),
          pltpu.SemaphoreType.DMA((NBUF,)),
          pltpu.SemaphoreType.DMA,
      ],
  )
  def k(ids_hbm, table_hbm, prompt_hbm, out_hbm, ids_v, idx_v, rows_v,
        prow_v, g_sems, o_sems, p_sem):
    wid = lax.axis_index("s") * NC + lax.axis_index("c")
    base = wid * per_w
    lane = lax.iota(jnp.int32, LANES)

    pltpu.sync_copy(ids_hbm.at[pl.ds(base, per_w)], ids_v)

    def p1(j, pacc):
      ids = ids_v[pl.ds(j * LANES, LANES)]
      is_p = ids >= VOCAB
      idx_v[pl.ds(j * LANES, LANES)] = jnp.where(is_p, REPLACING_ID, ids)
      return pacc + is_p.astype(jnp.int32)

    pacc = lax.fori_loop(0, n_groups, p1, jnp.zeros((LANES,), jnp.int32))
    n_prompt = jnp.sum(pacc)

    def _gather(c, b):
      return pltpu.make_async_copy(
          table_hbm.at[idx_v.at[pl.ds(c * CHUNK, CHUNK)]],
          rows_v.at[b], g_sems.at[b])

    def _out(c, b):
      return pltpu.make_async_copy(
          rows_v.at[b, :, pl.ds(0, D)],
          out_hbm.at[pl.ds(base + c * CHUNK, CHUNK)], o_sems.at[b])

    for c in range(n_chunks):
      b = c % NBUF
      if c >= NBUF:
        _out(c - NBUF, b).wait()
      _gather(c, b).start()
      if c >= 1:
        pb = (c - 1) % NBUF
        _gather(c - 1, pb).wait()
        _out(c - 1, pb).start()
    lb = (n_chunks - 1) % NBUF
    _gather(n_chunks - 1, lb).wait()
    _out(n_chunks - 1, lb).start()
    for c in range(n_chunks - NBUF + 1, n_chunks):
      _out(c, c % NBUF).wait()
    _out(n_chunks - NBUF, (n_chunks - NBUF) % NBUF).wait()

    @pl.when(n_prompt > 0)
    def _prompt_pass():
      @pl.loop(0, n_groups)
      def _p3(j):
        ids = ids_v[pl.ds(j * LANES, LANES)]
        is_p = ids >= VOCAB

        @pl.when(jnp.sum(is_p.astype(jnp.int32)) > 0)
        def _fixup():
          rel = jnp.minimum(ids - VOCAB, N_PROMPT - 1)
          pos = base + j * LANES + lane
          combo = jnp.where(is_p, pos * N_PROMPT + rel, -1)
          fill = jnp.max(combo)
          rel16 = jnp.where(is_p, rel, fill & (N_PROMPT - 1))
          pos16 = jnp.where(is_p, pos, fill >> 6)
          pltpu.async_copy(prompt_hbm.at[rel16], prow_v, p_sem).wait()
          pltpu.async_copy(prow_v, out_hbm.at[pos16], p_sem).wait()

  return k


def kernel(input_ids, labels, table, prompt_table):
  B, L = input_ids.shape
  n_tok = B * L
  ids_flat = input_ids.reshape(n_tok).astype(jnp.int32)
  table_p = jnp.pad(table, ((0, 0), (0, DP - D)))
  out = _make_kernel(n_tok)(ids_flat, table_p, prompt_table)
  return out.reshape(B, L, D)
